# baseline (speedup 1.0000x reference)
import dataclasses
import functools

import jax
import jax.numpy as jnp
from jax import lax
from jax.experimental import pallas as pl
from jax.experimental.pallas import tpu as pltpu
from jax.experimental.pallas import tpu_sc as plsc

N = 10000
E = 320000
F = 128
C = 40

NC = 2
NS = 16
NW = NC * NS
B = 128
CH = 80
EPW = CH * B
EPAD = NW * EPW
SUP = 20
NSUP = CH // SUP
NB = 2
GROUPS = SUP // NB
RPS = 624
TAIL = N - RPS * NS
LANES = 16


def _spmm_sc(src3, dst3, wflat, feats):
    mesh = plsc.VectorSubcoreMesh(core_axis_name="c", subcore_axis_name="s")
    cp = pltpu.CompilerParams()
    if "needs_layout_passes" in pltpu.CompilerParams.__dataclass_fields__:
        cp = dataclasses.replace(cp, needs_layout_passes=False)

    @functools.partial(
        pl.kernel,
        mesh=mesh,
        compiler_params=cp,
        out_type=jax.ShapeDtypeStruct((NC, N, F), jnp.float32),
        scratch_types=[
            pltpu.VMEM((SUP, B), jnp.int32),
            pltpu.VMEM((SUP, B), jnp.int32),
            pltpu.VMEM((SUP * B,), jnp.float32),
            pltpu.VMEM((B, F), jnp.float32),
            pltpu.VMEM_SHARED((N, F), jnp.float32),
        ],
    )
    def k(src_hbm, dst_hbm, w_hbm, feats_hbm, out_hbm, src_v, dst_v, w_v, buf, acc):
        c = lax.axis_index("c")
        s = lax.axis_index("s")
        wid = c * NS + s

        zero = jnp.zeros((LANES,), jnp.float32)
        for i in range(B):
            for cc in range(F // LANES):
                buf[i, pl.ds(cc * LANES, LANES)] = zero
        row0 = s * RPS
        for t in range(RPS // B):
            pltpu.sync_copy(buf, acc.at[pl.ds(row0 + t * B, B)])
        rem = RPS % B
        if rem:
            pltpu.sync_copy(buf.at[pl.ds(0, rem)],
                            acc.at[pl.ds(row0 + (RPS // B) * B, rem)])

        @pl.when(s == NS - 1)
        def _():
            pltpu.sync_copy(buf.at[pl.ds(0, TAIL)], acc.at[pl.ds(RPS * NS, TAIL)])

        plsc.subcore_barrier()

        @pl.loop(0, NSUP)
        def _(g):
            pltpu.sync_copy(src_hbm.at[wid * NSUP + g], src_v)
            pltpu.sync_copy(dst_hbm.at[wid * NSUP + g], dst_v)
            pltpu.sync_copy(w_hbm.at[wid * NSUP + g], w_v)

            @pl.loop(0, SUP)
            def _(kk):
                pltpu.sync_copy(feats_hbm.at[src_v.at[kk]], buf)
                base = kk * B
                for i in range(B):
                    wv = plsc.load_gather(
                        w_v, [jnp.full((LANES,), base + i, dtype=jnp.int32)])
                    for cc in range(F // LANES):
                        sl = (i, pl.ds(cc * LANES, LANES))
                        buf[sl] = buf[sl] * wv
                pltpu.sync_copy(buf, acc.at[dst_v.at[kk]], add=True)

        plsc.subcore_barrier()
        pltpu.sync_copy(acc.at[pl.ds(row0, RPS)], out_hbm.at[c, pl.ds(row0, RPS)])

        @pl.when(s == NS - 1)
        def _():
            pltpu.sync_copy(acc.at[pl.ds(RPS * NS, TAIL)],
                            out_hbm.at[c, pl.ds(RPS * NS, TAIL)])

    return k(src3, dst3, wflat, feats)


_ROWS = 2000


def _linear_tc(a, W, b):
    def body(a_ref, w_ref, b_ref, o_ref):
        o_ref[...] = (
            jnp.dot(a_ref[...], w_ref[...], preferred_element_type=jnp.float32)
            + b_ref[...]
        )

    return pl.pallas_call(
        body,
        grid=(N // _ROWS,),
        in_specs=[
            pl.BlockSpec((_ROWS, F), lambda i: (i, 0)),
            pl.BlockSpec((F, F), lambda i: (0, 0)),
            pl.BlockSpec((1, F), lambda i: (0, 0)),
        ],
        out_specs=pl.BlockSpec((_ROWS, F), lambda i: (i, 0)),
        out_shape=jax.ShapeDtypeStruct((N, F), jnp.float32),
    )(a, W, b.reshape(1, F))


def _fused_linear_tc(p, W, b):
    def body(pa_ref, pb_ref, w_ref, b_ref, o_ref):
        h = jnp.maximum(pa_ref[0] + pb_ref[0], 0.0)
        o_ref[...] = (
            jnp.dot(h, w_ref[...], preferred_element_type=jnp.float32)
            + b_ref[...]
        )

    return pl.pallas_call(
        body,
        grid=(N // _ROWS,),
        in_specs=[
            pl.BlockSpec((1, _ROWS, F), lambda i: (0, i, 0)),
            pl.BlockSpec((1, _ROWS, F), lambda i: (1, i, 0)),
            pl.BlockSpec((F, F), lambda i: (0, 0)),
            pl.BlockSpec((1, F), lambda i: (0, 0)),
        ],
        out_specs=pl.BlockSpec((_ROWS, F), lambda i: (i, 0)),
        out_shape=jax.ShapeDtypeStruct((N, F), jnp.float32),
    )(p, p, W, b.reshape(1, F))


def _final_tc(p, Wfc, bfc):
    def body(pa_ref, pb_ref, w_ref, b_ref, o_ref):
        z = jnp.maximum(pa_ref[0] + pb_ref[0], 0.0)
        logits = (
            jnp.dot(z, w_ref[...], preferred_element_type=jnp.float32)
            + b_ref[...]
        )
        m = jnp.max(logits, axis=1, keepdims=True)
        e = jnp.exp(logits - m)
        lse = jnp.log(jnp.sum(e, axis=1, keepdims=True))
        o_ref[...] = logits - m - lse

    return pl.pallas_call(
        body,
        grid=(N // _ROWS,),
        in_specs=[
            pl.BlockSpec((1, _ROWS, F), lambda i: (0, i, 0)),
            pl.BlockSpec((1, _ROWS, F), lambda i: (1, i, 0)),
            pl.BlockSpec((F, C), lambda i: (0, 0)),
            pl.BlockSpec((1, C), lambda i: (0, 0)),
        ],
        out_specs=pl.BlockSpec((_ROWS, C), lambda i: (i, 0)),
        out_shape=jax.ShapeDtypeStruct((N, C), jnp.float32),
    )(p, p, Wfc, bfc.reshape(1, C))


def kernel(x, edge_index, edge_weight, W1, b1, W2, b2, Wfc, bfc):
    npad = EPAD - E
    pad_idx = (jnp.arange(npad, dtype=jnp.int32) * 31) % N
    src = jnp.concatenate([edge_index[0].astype(jnp.int32), pad_idx])
    dst = jnp.concatenate([edge_index[1].astype(jnp.int32), pad_idx])
    w = jnp.concatenate(
        [edge_weight.astype(jnp.float32), jnp.zeros((npad,), jnp.float32)])
    src3 = src.reshape(NW * NSUP, SUP, B)
    dst3 = dst.reshape(NW * NSUP, SUP, B)
    wflat = w.reshape(NW * NSUP, SUP * B)

    s1 = _linear_tc(x, W1, b1)
    p1 = _spmm_sc(src3, dst3, wflat, s1)
    s2 = _fused_linear_tc(p1, W2, b2)
    p2 = _spmm_sc(src3, dst3, wflat, s2)
    return _final_tc(p2, Wfc, bfc)


# speedup vs baseline: 1.4616x; 1.4616x over previous
import dataclasses
import functools

import jax
import jax.numpy as jnp
from jax import lax
from jax.experimental import pallas as pl
from jax.experimental.pallas import tpu as pltpu
from jax.experimental.pallas import tpu_sc as plsc

N = 10000
E = 320000
F = 128
C = 40

NC = 2
NS = 16
NW = NC * NS
B = 80
CH = 125
EPW = CH * B
EPAD = NW * EPW
SUP = 25
NSUP = CH // SUP
RPS = 624
TAIL = N - RPS * NS
LANES = 16


def _spmm_sc(src3, dst3, wflat, feats):
    mesh = plsc.VectorSubcoreMesh(core_axis_name="c", subcore_axis_name="s")
    cp = pltpu.CompilerParams()
    if "needs_layout_passes" in pltpu.CompilerParams.__dataclass_fields__:
        cp = dataclasses.replace(cp, needs_layout_passes=False)

    @functools.partial(
        pl.kernel,
        mesh=mesh,
        compiler_params=cp,
        out_type=jax.ShapeDtypeStruct((NC, N, F), jnp.float32),
        scratch_types=[
            pltpu.VMEM((SUP, B), jnp.int32),
            pltpu.VMEM((SUP, B), jnp.int32),
            pltpu.VMEM((SUP * B,), jnp.float32),
            pltpu.VMEM((B, F), jnp.float32),
            pltpu.VMEM_SHARED((N, F), jnp.float32),
        ],
    )
    def k(src_hbm, dst_hbm, w_hbm, feats_hbm, out_hbm, src_v, dst_v, w_v, buf, acc):
        c = lax.axis_index("c")
        s = lax.axis_index("s")
        wid = c * NS + s

        zero = jnp.zeros((LANES,), jnp.float32)
        for i in range(B):
            for cc in range(F // LANES):
                buf[i, pl.ds(cc * LANES, LANES)] = zero
        row0 = s * RPS
        for t in range(RPS // B):
            pltpu.sync_copy(buf, acc.at[pl.ds(row0 + t * B, B)])
        rem = RPS % B
        if rem:
            pltpu.sync_copy(buf.at[pl.ds(0, rem)],
                            acc.at[pl.ds(row0 + (RPS // B) * B, rem)])

        @pl.when(s == NS - 1)
        def _():
            pltpu.sync_copy(buf.at[pl.ds(0, TAIL)], acc.at[pl.ds(RPS * NS, TAIL)])

        plsc.subcore_barrier()

        @pl.loop(0, NSUP)
        def _(g):
            pltpu.sync_copy(src_hbm.at[wid * NSUP + g], src_v)
            pltpu.sync_copy(dst_hbm.at[wid * NSUP + g], dst_v)
            pltpu.sync_copy(w_hbm.at[wid * NSUP + g], w_v)

            @pl.loop(0, SUP)
            def _(kk):
                pltpu.sync_copy(feats_hbm.at[src_v.at[kk]], buf)
                base = kk * B

                @plsc.parallel_loop(0, B, unroll=4)
                def _(i):
                    wv = plsc.load_gather(
                        w_v, [jnp.full((LANES,), base + i, dtype=jnp.int32)])
                    for cc in range(F // LANES):
                        sl = (i, pl.ds(cc * LANES, LANES))
                        buf[sl] = buf[sl] * wv
                pltpu.sync_copy(buf, acc.at[dst_v.at[kk]], add=True)

        plsc.subcore_barrier()
        pltpu.sync_copy(acc.at[pl.ds(row0, RPS)], out_hbm.at[c, pl.ds(row0, RPS)])

        @pl.when(s == NS - 1)
        def _():
            pltpu.sync_copy(acc.at[pl.ds(RPS * NS, TAIL)],
                            out_hbm.at[c, pl.ds(RPS * NS, TAIL)])

    return k(src3, dst3, wflat, feats)


_ROWS = 2000


def _linear_tc(a, W, b):
    def body(a_ref, w_ref, b_ref, o_ref):
        o_ref[...] = (
            jnp.dot(a_ref[...], w_ref[...], preferred_element_type=jnp.float32)
            + b_ref[...]
        )

    return pl.pallas_call(
        body,
        grid=(N // _ROWS,),
        in_specs=[
            pl.BlockSpec((_ROWS, F), lambda i: (i, 0)),
            pl.BlockSpec((F, F), lambda i: (0, 0)),
            pl.BlockSpec((1, F), lambda i: (0, 0)),
        ],
        out_specs=pl.BlockSpec((_ROWS, F), lambda i: (i, 0)),
        out_shape=jax.ShapeDtypeStruct((N, F), jnp.float32),
    )(a, W, b.reshape(1, F))


def _fused_linear_tc(p, W, b):
    def body(pa_ref, pb_ref, w_ref, b_ref, o_ref):
        h = jnp.maximum(pa_ref[0] + pb_ref[0], 0.0)
        o_ref[...] = (
            jnp.dot(h, w_ref[...], preferred_element_type=jnp.float32)
            + b_ref[...]
        )

    return pl.pallas_call(
        body,
        grid=(N // _ROWS,),
        in_specs=[
            pl.BlockSpec((1, _ROWS, F), lambda i: (0, i, 0)),
            pl.BlockSpec((1, _ROWS, F), lambda i: (1, i, 0)),
            pl.BlockSpec((F, F), lambda i: (0, 0)),
            pl.BlockSpec((1, F), lambda i: (0, 0)),
        ],
        out_specs=pl.BlockSpec((_ROWS, F), lambda i: (i, 0)),
        out_shape=jax.ShapeDtypeStruct((N, F), jnp.float32),
    )(p, p, W, b.reshape(1, F))


def _final_tc(p, Wfc, bfc):
    def body(pa_ref, pb_ref, w_ref, b_ref, o_ref):
        z = jnp.maximum(pa_ref[0] + pb_ref[0], 0.0)
        logits = (
            jnp.dot(z, w_ref[...], preferred_element_type=jnp.float32)
            + b_ref[...]
        )
        m = jnp.max(logits, axis=1, keepdims=True)
        e = jnp.exp(logits - m)
        lse = jnp.log(jnp.sum(e, axis=1, keepdims=True))
        o_ref[...] = logits - m - lse

    return pl.pallas_call(
        body,
        grid=(N // _ROWS,),
        in_specs=[
            pl.BlockSpec((1, _ROWS, F), lambda i: (0, i, 0)),
            pl.BlockSpec((1, _ROWS, F), lambda i: (1, i, 0)),
            pl.BlockSpec((F, C), lambda i: (0, 0)),
            pl.BlockSpec((1, C), lambda i: (0, 0)),
        ],
        out_specs=pl.BlockSpec((_ROWS, C), lambda i: (i, 0)),
        out_shape=jax.ShapeDtypeStruct((N, C), jnp.float32),
    )(p, p, Wfc, bfc.reshape(1, C))


def kernel(x, edge_index, edge_weight, W1, b1, W2, b2, Wfc, bfc):
    npad = EPAD - E
    pad_idx = (jnp.arange(npad, dtype=jnp.int32) * 31) % N
    src = jnp.concatenate([edge_index[0].astype(jnp.int32), pad_idx])
    dst = jnp.concatenate([edge_index[1].astype(jnp.int32), pad_idx])
    w = jnp.concatenate(
        [edge_weight.astype(jnp.float32), jnp.zeros((npad,), jnp.float32)])
    src3 = src.reshape(NW * NSUP, SUP, B)
    dst3 = dst.reshape(NW * NSUP, SUP, B)
    wflat = w.reshape(NW * NSUP, SUP * B)

    s1 = _linear_tc(x, W1, b1)
    p1 = _spmm_sc(src3, dst3, wflat, s1)
    s2 = _fused_linear_tc(p1, W2, b2)
    p2 = _spmm_sc(src3, dst3, wflat, s2)
    return _final_tc(p2, Wfc, bfc)


# speedup vs baseline: 2.4430x; 1.6715x over previous
import dataclasses
import functools

import jax
import jax.numpy as jnp
from jax import lax
from jax.experimental import pallas as pl
from jax.experimental.pallas import tpu as pltpu
from jax.experimental.pallas import tpu_sc as plsc

N = 10000
E = 320000
F = 128
C = 40

NC = 2
NS = 16
NW = NC * NS
B = 80
CH = 126
EPW = CH * B
EPAD = NW * EPW
SUP = 21
NSUP = CH // SUP
NB = 3
GROUPS = SUP // NB
RPS = 624
TAIL = N - RPS * NS
LANES = 16


def _spmm_sc(src3, dst3, wflat, feats):
    mesh = plsc.VectorSubcoreMesh(core_axis_name="c", subcore_axis_name="s")
    cp = pltpu.CompilerParams()
    if "needs_layout_passes" in pltpu.CompilerParams.__dataclass_fields__:
        cp = dataclasses.replace(cp, needs_layout_passes=False)

    @functools.partial(
        pl.kernel,
        mesh=mesh,
        compiler_params=cp,
        out_type=jax.ShapeDtypeStruct((NC, N, F), jnp.float32),
        scratch_types=[
            pltpu.VMEM((SUP, B), jnp.int32),
            pltpu.VMEM((SUP, B), jnp.int32),
            pltpu.VMEM((SUP * B,), jnp.float32),
            pltpu.VMEM((B, F), jnp.float32),
            pltpu.VMEM((B, F), jnp.float32),
            pltpu.VMEM((B, F), jnp.float32),
            pltpu.VMEM_SHARED((N, F), jnp.float32),
            pltpu.SemaphoreType.DMA,
            pltpu.SemaphoreType.DMA,
            pltpu.SemaphoreType.DMA,
            pltpu.SemaphoreType.DMA,
            pltpu.SemaphoreType.DMA,
            pltpu.SemaphoreType.DMA,
        ],
    )
    def k(src_hbm, dst_hbm, w_hbm, feats_hbm, out_hbm, src_v, dst_v, w_v,
          buf0, buf1, buf2, acc, g0, g1, g2, w0, w1, w2):
        c = lax.axis_index("c")
        s = lax.axis_index("s")
        wid = c * NS + s
        bufs = [buf0, buf1, buf2]
        gsems = [g0, g1, g2]
        wsems = [w0, w1, w2]
        buf = buf0

        def issue_gather(b, kk):
            pltpu.async_copy(feats_hbm.at[src_v.at[kk]], bufs[b], gsems[b])

        def wait_gather(b):
            pltpu.make_async_copy(feats_hbm.at[src_v.at[0]], bufs[b],
                                  gsems[b]).wait()

        def issue_scatter(b, kk):
            pltpu.async_copy(bufs[b], acc.at[dst_v.at[kk]], wsems[b], add=True)

        def wait_scatter(b):
            pltpu.make_async_copy(bufs[b], acc.at[dst_v.at[0]], wsems[b]).wait()

        def scale(b, kk):
            bb = bufs[b]
            base = kk * B

            @plsc.parallel_loop(0, B, unroll=4)
            def _(i):
                wv = plsc.load_gather(
                    w_v, [jnp.full((LANES,), base + i, dtype=jnp.int32)])
                for cc in range(F // LANES):
                    sl = (i, pl.ds(cc * LANES, LANES))
                    bb[sl] = bb[sl] * wv

        zero = jnp.zeros((LANES,), jnp.float32)
        for i in range(B):
            for cc in range(F // LANES):
                buf[i, pl.ds(cc * LANES, LANES)] = zero
        row0 = s * RPS
        for t in range(RPS // B):
            pltpu.sync_copy(buf, acc.at[pl.ds(row0 + t * B, B)])
        rem = RPS % B
        if rem:
            pltpu.sync_copy(buf.at[pl.ds(0, rem)],
                            acc.at[pl.ds(row0 + (RPS // B) * B, rem)])

        @pl.when(s == NS - 1)
        def _():
            pltpu.sync_copy(buf.at[pl.ds(0, TAIL)], acc.at[pl.ds(RPS * NS, TAIL)])

        plsc.subcore_barrier()

        @pl.loop(0, NSUP)
        def _(g):
            pltpu.sync_copy(src_hbm.at[wid * NSUP + g], src_v)
            pltpu.sync_copy(dst_hbm.at[wid * NSUP + g], dst_v)
            pltpu.sync_copy(w_hbm.at[wid * NSUP + g], w_v)

            issue_gather(0, 0)
            issue_gather(1, 1)

            @pl.loop(0, GROUPS)
            def _(j0):
                for b in range(NB):
                    kk = j0 * NB + b
                    nxt = kk + 2
                    bn = (b + 2) % NB

                    @pl.when(kk == 0)
                    def _():
                        issue_gather(bn, nxt)

                    @pl.when((kk >= 1) & (nxt < SUP))
                    def _():
                        wait_scatter(bn)
                        issue_gather(bn, nxt)

                    wait_gather(b)
                    scale(b, kk)
                    issue_scatter(b, kk)

            for b in range(NB):
                wait_scatter(b)

        plsc.subcore_barrier()
        pltpu.sync_copy(acc.at[pl.ds(row0, RPS)], out_hbm.at[c, pl.ds(row0, RPS)])

        @pl.when(s == NS - 1)
        def _():
            pltpu.sync_copy(acc.at[pl.ds(RPS * NS, TAIL)],
                            out_hbm.at[c, pl.ds(RPS * NS, TAIL)])

    return k(src3, dst3, wflat, feats)


_ROWS = 2000


def _linear_tc(a, W, b):
    def body(a_ref, w_ref, b_ref, o_ref):
        o_ref[...] = (
            jnp.dot(a_ref[...], w_ref[...], preferred_element_type=jnp.float32)
            + b_ref[...]
        )

    return pl.pallas_call(
        body,
        grid=(N // _ROWS,),
        in_specs=[
            pl.BlockSpec((_ROWS, F), lambda i: (i, 0)),
            pl.BlockSpec((F, F), lambda i: (0, 0)),
            pl.BlockSpec((1, F), lambda i: (0, 0)),
        ],
        out_specs=pl.BlockSpec((_ROWS, F), lambda i: (i, 0)),
        out_shape=jax.ShapeDtypeStruct((N, F), jnp.float32),
    )(a, W, b.reshape(1, F))


def _fused_linear_tc(p, W, b):
    def body(pa_ref, pb_ref, w_ref, b_ref, o_ref):
        h = jnp.maximum(pa_ref[0] + pb_ref[0], 0.0)
        o_ref[...] = (
            jnp.dot(h, w_ref[...], preferred_element_type=jnp.float32)
            + b_ref[...]
        )

    return pl.pallas_call(
        body,
        grid=(N // _ROWS,),
        in_specs=[
            pl.BlockSpec((1, _ROWS, F), lambda i: (0, i, 0)),
            pl.BlockSpec((1, _ROWS, F), lambda i: (1, i, 0)),
            pl.BlockSpec((F, F), lambda i: (0, 0)),
            pl.BlockSpec((1, F), lambda i: (0, 0)),
        ],
        out_specs=pl.BlockSpec((_ROWS, F), lambda i: (i, 0)),
        out_shape=jax.ShapeDtypeStruct((N, F), jnp.float32),
    )(p, p, W, b.reshape(1, F))


def _final_tc(p, Wfc, bfc):
    def body(pa_ref, pb_ref, w_ref, b_ref, o_ref):
        z = jnp.maximum(pa_ref[0] + pb_ref[0], 0.0)
        logits = (
            jnp.dot(z, w_ref[...], preferred_element_type=jnp.float32)
            + b_ref[...]
        )
        m = jnp.max(logits, axis=1, keepdims=True)
        e = jnp.exp(logits - m)
        lse = jnp.log(jnp.sum(e, axis=1, keepdims=True))
        o_ref[...] = logits - m - lse

    return pl.pallas_call(
        body,
        grid=(N // _ROWS,),
        in_specs=[
            pl.BlockSpec((1, _ROWS, F), lambda i: (0, i, 0)),
            pl.BlockSpec((1, _ROWS, F), lambda i: (1, i, 0)),
            pl.BlockSpec((F, C), lambda i: (0, 0)),
            pl.BlockSpec((1, C), lambda i: (0, 0)),
        ],
        out_specs=pl.BlockSpec((_ROWS, C), lambda i: (i, 0)),
        out_shape=jax.ShapeDtypeStruct((N, C), jnp.float32),
    )(p, p, Wfc, bfc.reshape(1, C))


def kernel(x, edge_index, edge_weight, W1, b1, W2, b2, Wfc, bfc):
    npad = EPAD - E
    pad_idx = (jnp.arange(npad, dtype=jnp.int32) * 31) % N
    src = jnp.concatenate([edge_index[0].astype(jnp.int32), pad_idx])
    dst = jnp.concatenate([edge_index[1].astype(jnp.int32), pad_idx])
    w = jnp.concatenate(
        [edge_weight.astype(jnp.float32), jnp.zeros((npad,), jnp.float32)])
    src3 = src.reshape(NW * NSUP, SUP, B)
    dst3 = dst.reshape(NW * NSUP, SUP, B)
    wflat = w.reshape(NW * NSUP, SUP * B)

    s1 = _linear_tc(x, W1, b1)
    p1 = _spmm_sc(src3, dst3, wflat, s1)
    s2 = _fused_linear_tc(p1, W2, b2)
    p2 = _spmm_sc(src3, dst3, wflat, s2)
    return _final_tc(p2, Wfc, bfc)


# speedup vs baseline: 2.7766x; 1.1366x over previous
import dataclasses
import functools

import jax
import jax.numpy as jnp
from jax import lax
from jax.experimental import pallas as pl
from jax.experimental.pallas import tpu as pltpu
from jax.experimental.pallas import tpu_sc as plsc

N = 10000
E = 320000
F = 128
C = 40

NC = 2
NS = 16
NW = NC * NS
B = 80
CH = 126
EPW = CH * B
EPAD = NW * EPW
SUP = 9
NSUP = CH // SUP
NB = 3
GROUPS = SUP // NB
RPS = 624
TAIL = N - RPS * NS
LANES = 16


def _spmm_sc(src3, dst3, wflat, feats):
    mesh = plsc.VectorSubcoreMesh(core_axis_name="c", subcore_axis_name="s")
    cp = pltpu.CompilerParams()
    if "needs_layout_passes" in pltpu.CompilerParams.__dataclass_fields__:
        cp = dataclasses.replace(cp, needs_layout_passes=False)

    @functools.partial(
        pl.kernel,
        mesh=mesh,
        compiler_params=cp,
        out_type=jax.ShapeDtypeStruct((NC, N, F), jnp.float32),
        scratch_types=[
            pltpu.VMEM((2, SUP, B), jnp.int32),
            pltpu.VMEM((2, SUP, B), jnp.int32),
            pltpu.VMEM((2, SUP * B), jnp.float32),
            pltpu.VMEM((B, F), jnp.float32),
            pltpu.VMEM((B, F), jnp.float32),
            pltpu.VMEM((B, F), jnp.float32),
            pltpu.VMEM_SHARED((N, F), jnp.float32),
            pltpu.SemaphoreType.DMA,
            pltpu.SemaphoreType.DMA,
            pltpu.SemaphoreType.DMA,
            pltpu.SemaphoreType.DMA,
            pltpu.SemaphoreType.DMA,
            pltpu.SemaphoreType.DMA,
            pltpu.SemaphoreType.DMA,
        ],
    )
    def k(src_hbm, dst_hbm, w_hbm, feats_hbm, out_hbm, src_v, dst_v, w_v,
          buf0, buf1, buf2, acc, g0, g1, g2, w0, w1, w2, ssem):
        c = lax.axis_index("c")
        s = lax.axis_index("s")
        wid = c * NS + s
        bufs = [buf0, buf1, buf2]
        gsems = [g0, g1, g2]
        wsems = [w0, w1, w2]
        buf = buf0

        def stage(slot, gg):
            pltpu.async_copy(src_hbm.at[wid * NSUP + gg], src_v.at[slot], ssem)
            pltpu.async_copy(dst_hbm.at[wid * NSUP + gg], dst_v.at[slot], ssem)
            pltpu.async_copy(w_hbm.at[wid * NSUP + gg], w_v.at[slot], ssem)

        def wait_stage():
            pltpu.make_async_copy(src_hbm.at[0], src_v.at[0], ssem).wait()
            pltpu.make_async_copy(dst_hbm.at[0], dst_v.at[0], ssem).wait()
            pltpu.make_async_copy(w_hbm.at[0], w_v.at[0], ssem).wait()

        def issue_gather(b, slot, kk):
            pltpu.async_copy(feats_hbm.at[src_v.at[slot, kk]], bufs[b],
                             gsems[b])

        def wait_gather(b):
            pltpu.make_async_copy(feats_hbm.at[src_v.at[0, 0]], bufs[b],
                                  gsems[b]).wait()

        def issue_scatter(b, slot, kk):
            pltpu.async_copy(bufs[b], acc.at[dst_v.at[slot, kk]], wsems[b],
                             add=True)

        def wait_scatter(b):
            pltpu.make_async_copy(bufs[b], acc.at[dst_v.at[0, 0]],
                                  wsems[b]).wait()

        def scale(b, slot, kk):
            bb = bufs[b]
            base = kk * B
            slot_vec = jnp.full((LANES,), slot, dtype=jnp.int32)

            @plsc.parallel_loop(0, B, unroll=8)
            def _(i):
                wv = plsc.load_gather(
                    w_v,
                    [slot_vec, jnp.full((LANES,), base + i, dtype=jnp.int32)])
                for cc in range(F // LANES):
                    sl = (i, pl.ds(cc * LANES, LANES))
                    bb[sl] = bb[sl] * wv

        zero = jnp.zeros((LANES,), jnp.float32)

        @plsc.parallel_loop(0, B, unroll=4)
        def _(i):
            for cc in range(F // LANES):
                buf[i, pl.ds(cc * LANES, LANES)] = zero
        row0 = s * RPS
        for t in range(RPS // B):
            pltpu.sync_copy(buf, acc.at[pl.ds(row0 + t * B, B)])
        rem = RPS % B
        if rem:
            pltpu.sync_copy(buf.at[pl.ds(0, rem)],
                            acc.at[pl.ds(row0 + (RPS // B) * B, rem)])

        @pl.when(s == NS - 1)
        def _():
            pltpu.sync_copy(buf.at[pl.ds(0, TAIL)], acc.at[pl.ds(RPS * NS, TAIL)])

        plsc.subcore_barrier()

        stage(0, 0)
        wait_stage()
        issue_gather(0, 0, 0)
        issue_gather(1, 0, 1)

        @pl.loop(0, NSUP)
        def _(g):
            slot = g % 2
            nslot = 1 - slot

            @pl.loop(0, GROUPS)
            def _(j0):
                @pl.when((j0 == 1) & (g + 1 < NSUP))
                def _():
                    stage(nslot, g + 1)

                for b in range(NB):
                    kk = j0 * NB + b
                    nxt = kk + 2
                    bn = (b + 2) % NB

                    @pl.when(nxt < SUP)
                    def _():
                        @pl.when((g > 0) | (kk >= 1))
                        def _():
                            wait_scatter(bn)

                        issue_gather(bn, slot, nxt)

                    @pl.when((nxt >= SUP) & (g + 1 < NSUP))
                    def _():
                        @pl.when(nxt == SUP)
                        def _():
                            wait_stage()

                        wait_scatter(bn)
                        issue_gather(bn, nslot, nxt - SUP)

                    wait_gather(b)
                    scale(b, slot, kk)
                    issue_scatter(b, slot, kk)

        for b in range(NB):
            wait_scatter(b)

        plsc.subcore_barrier()
        pltpu.sync_copy(acc.at[pl.ds(row0, RPS)], out_hbm.at[c, pl.ds(row0, RPS)])

        @pl.when(s == NS - 1)
        def _():
            pltpu.sync_copy(acc.at[pl.ds(RPS * NS, TAIL)],
                            out_hbm.at[c, pl.ds(RPS * NS, TAIL)])

    return k(src3, dst3, wflat, feats)


_ROWS = 2000


def _linear_tc(a, W, b):
    def body(a_ref, w_ref, b_ref, o_ref):
        o_ref[...] = (
            jnp.dot(a_ref[...], w_ref[...], preferred_element_type=jnp.float32)
            + b_ref[...]
        )

    return pl.pallas_call(
        body,
        grid=(N // _ROWS,),
        in_specs=[
            pl.BlockSpec((_ROWS, F), lambda i: (i, 0)),
            pl.BlockSpec((F, F), lambda i: (0, 0)),
            pl.BlockSpec((1, F), lambda i: (0, 0)),
        ],
        out_specs=pl.BlockSpec((_ROWS, F), lambda i: (i, 0)),
        out_shape=jax.ShapeDtypeStruct((N, F), jnp.float32),
    )(a, W, b.reshape(1, F))


def _fused_linear_tc(p, W, b):
    def body(pa_ref, pb_ref, w_ref, b_ref, o_ref):
        h = jnp.maximum(pa_ref[0] + pb_ref[0], 0.0)
        o_ref[...] = (
            jnp.dot(h, w_ref[...], preferred_element_type=jnp.float32)
            + b_ref[...]
        )

    return pl.pallas_call(
        body,
        grid=(N // _ROWS,),
        in_specs=[
            pl.BlockSpec((1, _ROWS, F), lambda i: (0, i, 0)),
            pl.BlockSpec((1, _ROWS, F), lambda i: (1, i, 0)),
            pl.BlockSpec((F, F), lambda i: (0, 0)),
            pl.BlockSpec((1, F), lambda i: (0, 0)),
        ],
        out_specs=pl.BlockSpec((_ROWS, F), lambda i: (i, 0)),
        out_shape=jax.ShapeDtypeStruct((N, F), jnp.float32),
    )(p, p, W, b.reshape(1, F))


def _final_tc(p, Wfc, bfc):
    def body(pa_ref, pb_ref, w_ref, b_ref, o_ref):
        z = jnp.maximum(pa_ref[0] + pb_ref[0], 0.0)
        logits = (
            jnp.dot(z, w_ref[...], preferred_element_type=jnp.float32)
            + b_ref[...]
        )
        m = jnp.max(logits, axis=1, keepdims=True)
        e = jnp.exp(logits - m)
        lse = jnp.log(jnp.sum(e, axis=1, keepdims=True))
        o_ref[...] = logits - m - lse

    return pl.pallas_call(
        body,
        grid=(N // _ROWS,),
        in_specs=[
            pl.BlockSpec((1, _ROWS, F), lambda i: (0, i, 0)),
            pl.BlockSpec((1, _ROWS, F), lambda i: (1, i, 0)),
            pl.BlockSpec((F, C), lambda i: (0, 0)),
            pl.BlockSpec((1, C), lambda i: (0, 0)),
        ],
        out_specs=pl.BlockSpec((_ROWS, C), lambda i: (i, 0)),
        out_shape=jax.ShapeDtypeStruct((N, C), jnp.float32),
    )(p, p, Wfc, bfc.reshape(1, C))


def kernel(x, edge_index, edge_weight, W1, b1, W2, b2, Wfc, bfc):
    npad = EPAD - E
    pad_idx = (jnp.arange(npad, dtype=jnp.int32) * 31) % N
    src = jnp.concatenate([edge_index[0].astype(jnp.int32), pad_idx])
    dst = jnp.concatenate([edge_index[1].astype(jnp.int32), pad_idx])
    w = jnp.concatenate(
        [edge_weight.astype(jnp.float32), jnp.zeros((npad,), jnp.float32)])
    src3 = src.reshape(NW * NSUP, SUP, B)
    dst3 = dst.reshape(NW * NSUP, SUP, B)
    wflat = w.reshape(NW * NSUP, SUP * B)

    s1 = _linear_tc(x, W1, b1)
    p1 = _spmm_sc(src3, dst3, wflat, s1)
    s2 = _fused_linear_tc(p1, W2, b2)
    p2 = _spmm_sc(src3, dst3, wflat, s2)
    return _final_tc(p2, Wfc, bfc)


# speedup vs baseline: 2.7796x; 1.0011x over previous
import dataclasses
import functools

import jax
import jax.numpy as jnp
from jax import lax
from jax.experimental import pallas as pl
from jax.experimental.pallas import tpu as pltpu
from jax.experimental.pallas import tpu_sc as plsc

N = 10000
E = 320000
F = 128
C = 40

NC = 2
NS = 16
NW = NC * NS
B = 80
CH = 126
EPW = CH * B
EPAD = NW * EPW
SUP = 18
NSUP = CH // SUP
NB = 3
GROUPS = SUP // NB
RPS = 624
TAIL = N - RPS * NS
LANES = 16


def _spmm_sc(src3, dst3, wflat, feats):
    mesh = plsc.VectorSubcoreMesh(core_axis_name="c", subcore_axis_name="s")
    cp = pltpu.CompilerParams()
    if "needs_layout_passes" in pltpu.CompilerParams.__dataclass_fields__:
        cp = dataclasses.replace(cp, needs_layout_passes=False)

    @functools.partial(
        pl.kernel,
        mesh=mesh,
        compiler_params=cp,
        out_type=jax.ShapeDtypeStruct((NC, N, F), jnp.float32),
        scratch_types=[
            pltpu.VMEM((2, SUP, B), jnp.int32),
            pltpu.VMEM((2, SUP, B), jnp.int32),
            pltpu.VMEM((2, SUP * B), jnp.float32),
            pltpu.VMEM((B, F), jnp.float32),
            pltpu.VMEM((B, F), jnp.float32),
            pltpu.VMEM((B, F), jnp.float32),
            pltpu.VMEM_SHARED((N, F), jnp.float32),
            pltpu.SemaphoreType.DMA,
            pltpu.SemaphoreType.DMA,
            pltpu.SemaphoreType.DMA,
            pltpu.SemaphoreType.DMA,
            pltpu.SemaphoreType.DMA,
            pltpu.SemaphoreType.DMA,
            pltpu.SemaphoreType.DMA,
        ],
    )
    def k(src_hbm, dst_hbm, w_hbm, feats_hbm, out_hbm, src_v, dst_v, w_v,
          buf0, buf1, buf2, acc, g0, g1, g2, w0, w1, w2, ssem):
        c = lax.axis_index("c")
        s = lax.axis_index("s")
        wid = c * NS + s
        bufs = [buf0, buf1, buf2]
        gsems = [g0, g1, g2]
        wsems = [w0, w1, w2]
        buf = buf0

        def stage(slot, gg):
            pltpu.async_copy(src_hbm.at[wid * NSUP + gg], src_v.at[slot], ssem)
            pltpu.async_copy(dst_hbm.at[wid * NSUP + gg], dst_v.at[slot], ssem)
            pltpu.async_copy(w_hbm.at[wid * NSUP + gg], w_v.at[slot], ssem)

        def wait_stage():
            pltpu.make_async_copy(src_hbm.at[0], src_v.at[0], ssem).wait()
            pltpu.make_async_copy(dst_hbm.at[0], dst_v.at[0], ssem).wait()
            pltpu.make_async_copy(w_hbm.at[0], w_v.at[0], ssem).wait()

        def issue_gather(b, slot, kk):
            pltpu.async_copy(feats_hbm.at[src_v.at[slot, kk]], bufs[b],
                             gsems[b])

        def wait_gather(b):
            pltpu.make_async_copy(feats_hbm.at[src_v.at[0, 0]], bufs[b],
                                  gsems[b]).wait()

        def issue_scatter(b, slot, kk):
            pltpu.async_copy(bufs[b], acc.at[dst_v.at[slot, kk]], wsems[b],
                             add=True)

        def wait_scatter(b):
            pltpu.make_async_copy(bufs[b], acc.at[dst_v.at[0, 0]],
                                  wsems[b]).wait()

        def scale(b, slot, kk):
            bb = bufs[b]
            base = kk * B
            slot_vec = jnp.full((LANES,), slot, dtype=jnp.int32)

            @plsc.parallel_loop(0, B, unroll=8)
            def _(i):
                wv = plsc.load_gather(
                    w_v,
                    [slot_vec, jnp.full((LANES,), base + i, dtype=jnp.int32)])
                for cc in range(F // LANES):
                    sl = (i, pl.ds(cc * LANES, LANES))
                    bb[sl] = bb[sl] * wv

        zero = jnp.zeros((LANES,), jnp.float32)

        @plsc.parallel_loop(0, B, unroll=4)
        def _(i):
            for cc in range(F // LANES):
                buf[i, pl.ds(cc * LANES, LANES)] = zero
        row0 = s * RPS
        for t in range(RPS // B):
            pltpu.sync_copy(buf, acc.at[pl.ds(row0 + t * B, B)])
        rem = RPS % B
        if rem:
            pltpu.sync_copy(buf.at[pl.ds(0, rem)],
                            acc.at[pl.ds(row0 + (RPS // B) * B, rem)])

        @pl.when(s == NS - 1)
        def _():
            pltpu.sync_copy(buf.at[pl.ds(0, TAIL)], acc.at[pl.ds(RPS * NS, TAIL)])

        plsc.subcore_barrier()

        stage(0, 0)
        wait_stage()
        issue_gather(0, 0, 0)
        issue_gather(1, 0, 1)

        @pl.loop(0, NSUP)
        def _(g):
            slot = g % 2
            nslot = 1 - slot

            @pl.loop(0, GROUPS)
            def _(j0):
                @pl.when((j0 == 1) & (g + 1 < NSUP))
                def _():
                    stage(nslot, g + 1)

                for b in range(NB):
                    kk = j0 * NB + b
                    nxt = kk + 2
                    bn = (b + 2) % NB

                    @pl.when(nxt < SUP)
                    def _():
                        @pl.when((g > 0) | (kk >= 1))
                        def _():
                            wait_scatter(bn)

                        issue_gather(bn, slot, nxt)

                    @pl.when((nxt >= SUP) & (g + 1 < NSUP))
                    def _():
                        @pl.when(nxt == SUP)
                        def _():
                            wait_stage()

                        wait_scatter(bn)
                        issue_gather(bn, nslot, nxt - SUP)

                    wait_gather(b)
                    scale(b, slot, kk)
                    issue_scatter(b, slot, kk)

        for b in range(NB):
            wait_scatter(b)

        plsc.subcore_barrier()
        pltpu.sync_copy(acc.at[pl.ds(row0, RPS)], out_hbm.at[c, pl.ds(row0, RPS)])

        @pl.when(s == NS - 1)
        def _():
            pltpu.sync_copy(acc.at[pl.ds(RPS * NS, TAIL)],
                            out_hbm.at[c, pl.ds(RPS * NS, TAIL)])

    return k(src3, dst3, wflat, feats)


_ROWS = 2000


def _linear_tc(a, W, b):
    def body(a_ref, w_ref, b_ref, o_ref):
        o_ref[...] = (
            jnp.dot(a_ref[...], w_ref[...], preferred_element_type=jnp.float32)
            + b_ref[...]
        )

    return pl.pallas_call(
        body,
        grid=(N // _ROWS,),
        in_specs=[
            pl.BlockSpec((_ROWS, F), lambda i: (i, 0)),
            pl.BlockSpec((F, F), lambda i: (0, 0)),
            pl.BlockSpec((1, F), lambda i: (0, 0)),
        ],
        out_specs=pl.BlockSpec((_ROWS, F), lambda i: (i, 0)),
        out_shape=jax.ShapeDtypeStruct((N, F), jnp.float32),
    )(a, W, b.reshape(1, F))


def _fused_linear_tc(p, W, b):
    def body(pa_ref, pb_ref, w_ref, b_ref, o_ref):
        h = jnp.maximum(pa_ref[0] + pb_ref[0], 0.0)
        o_ref[...] = (
            jnp.dot(h, w_ref[...], preferred_element_type=jnp.float32)
            + b_ref[...]
        )

    return pl.pallas_call(
        body,
        grid=(N // _ROWS,),
        in_specs=[
            pl.BlockSpec((1, _ROWS, F), lambda i: (0, i, 0)),
            pl.BlockSpec((1, _ROWS, F), lambda i: (1, i, 0)),
            pl.BlockSpec((F, F), lambda i: (0, 0)),
            pl.BlockSpec((1, F), lambda i: (0, 0)),
        ],
        out_specs=pl.BlockSpec((_ROWS, F), lambda i: (i, 0)),
        out_shape=jax.ShapeDtypeStruct((N, F), jnp.float32),
    )(p, p, W, b.reshape(1, F))


def _final_tc(p, Wfc, bfc):
    def body(pa_ref, pb_ref, w_ref, b_ref, o_ref):
        z = jnp.maximum(pa_ref[0] + pb_ref[0], 0.0)
        logits = (
            jnp.dot(z, w_ref[...], preferred_element_type=jnp.float32)
            + b_ref[...]
        )
        m = jnp.max(logits, axis=1, keepdims=True)
        e = jnp.exp(logits - m)
        lse = jnp.log(jnp.sum(e, axis=1, keepdims=True))
        o_ref[...] = logits - m - lse

    return pl.pallas_call(
        body,
        grid=(N // _ROWS,),
        in_specs=[
            pl.BlockSpec((1, _ROWS, F), lambda i: (0, i, 0)),
            pl.BlockSpec((1, _ROWS, F), lambda i: (1, i, 0)),
            pl.BlockSpec((F, C), lambda i: (0, 0)),
            pl.BlockSpec((1, C), lambda i: (0, 0)),
        ],
        out_specs=pl.BlockSpec((_ROWS, C), lambda i: (i, 0)),
        out_shape=jax.ShapeDtypeStruct((N, C), jnp.float32),
    )(p, p, Wfc, bfc.reshape(1, C))


def kernel(x, edge_index, edge_weight, W1, b1, W2, b2, Wfc, bfc):
    npad = EPAD - E
    pad_idx = (jnp.arange(npad, dtype=jnp.int32) * 31) % N
    src = jnp.concatenate([edge_index[0].astype(jnp.int32), pad_idx])
    dst = jnp.concatenate([edge_index[1].astype(jnp.int32), pad_idx])
    w = jnp.concatenate(
        [edge_weight.astype(jnp.float32), jnp.zeros((npad,), jnp.float32)])
    src3 = src.reshape(NW * NSUP, SUP, B)
    dst3 = dst.reshape(NW * NSUP, SUP, B)
    wflat = w.reshape(NW * NSUP, SUP * B)

    s1 = _linear_tc(x, W1, b1)
    p1 = _spmm_sc(src3, dst3, wflat, s1)
    s2 = _fused_linear_tc(p1, W2, b2)
    p2 = _spmm_sc(src3, dst3, wflat, s2)
    return _final_tc(p2, Wfc, bfc)
